# Initial kernel scaffold; baseline (speedup 1.0000x reference)
import jax
import jax.numpy as jnp
from jax.experimental import pallas as pl


def kernel(x, edge_index, W1l, W1r, b1, W2l, W2r, b2, W3l, W3r, b3):
    raise NotImplementedError("write your pallas kernel here")


# speedup vs baseline: 4.7403x; 4.7403x over previous
import functools

import jax
import jax.numpy as jnp
from jax import lax
from jax.experimental import pallas as pl
from jax.experimental.pallas import tpu as pltpu
from jax.experimental.pallas import tpu_sc as plsc

N_NODES = 10000
N_PAD = 10240
N_EDGES = 320000
D = 128
NC = 2
NS = 16
NW = NC * NS
EPT = N_EDGES // NW
CH = 80
NCHUNK = EPT // CH
ROWS_PT = N_PAD // NS

_MESH = plsc.VectorSubcoreMesh(core_axis_name="c", subcore_axis_name="s")


@functools.partial(
    pl.kernel, mesh=_MESH,
    out_type=jax.ShapeDtypeStruct((NC * N_PAD, D), jnp.float32),
    scratch_types=[
        pltpu.VMEM((CH,), jnp.int32),
        pltpu.VMEM((CH,), jnp.int32),
        pltpu.VMEM((CH, D), jnp.float32),
        pltpu.VMEM_SHARED((N_PAD, D), jnp.float32),
        pltpu.SemaphoreType.DMA,
    ],
)
def _segsum(table, src, dst, z128, out, src_v, dst_v, rows_v, acc_sh, sem):
    cid = lax.axis_index("c")
    sid = lax.axis_index("s")
    rs = sid * ROWS_PT
    pltpu.sync_copy(z128.at[pl.ds(0, CH)], rows_v)
    for j in range(ROWS_PT // CH):
        pltpu.sync_copy(rows_v, acc_sh.at[pl.ds(rs + j * CH, CH)])
    plsc.subcore_barrier()

    base = (cid * NS + sid) * EPT

    def body(ci, carry):
        off = base + ci * CH
        pltpu.sync_copy(src.at[pl.ds(off, CH)], src_v)
        pltpu.sync_copy(dst.at[pl.ds(off, CH)], dst_v)
        pltpu.async_copy(table.at[src_v], rows_v, sem).wait()
        pltpu.sync_copy(rows_v, acc_sh.at[dst_v], add=True)
        return carry

    lax.fori_loop(0, NCHUNK, body, 0)
    plsc.subcore_barrier()
    obase = cid * N_PAD + rs
    for j in range(ROWS_PT // CH):
        pltpu.sync_copy(acc_sh.at[pl.ds(rs + j * CH, CH)], rows_v)
        pltpu.sync_copy(rows_v, out.at[pl.ds(obase + j * CH, CH)])


@functools.partial(
    pl.kernel, mesh=_MESH,
    out_type=jax.ShapeDtypeStruct((NC * N_PAD, D), jnp.float32),
    scratch_types=[
        pltpu.VMEM((CH,), jnp.int32),
        pltpu.VMEM((CH, D), jnp.float32),
        pltpu.VMEM_SHARED((N_PAD, D), jnp.float32),
    ],
)
def _cnt128(dst, z128, ones, out, dst_v, rows_v, acc_sh):
    cid = lax.axis_index("c")
    sid = lax.axis_index("s")
    rs = sid * ROWS_PT
    pltpu.sync_copy(z128.at[pl.ds(0, CH)], rows_v)
    for j in range(ROWS_PT // CH):
        pltpu.sync_copy(rows_v, acc_sh.at[pl.ds(rs + j * CH, CH)])
    pltpu.sync_copy(ones, rows_v)
    plsc.subcore_barrier()

    base = (cid * NS + sid) * EPT

    def body(ci, carry):
        off = base + ci * CH
        pltpu.sync_copy(dst.at[pl.ds(off, CH)], dst_v)
        pltpu.sync_copy(rows_v, acc_sh.at[dst_v], add=True)
        return carry

    lax.fori_loop(0, NCHUNK, body, 0)
    plsc.subcore_barrier()
    obase = cid * N_PAD + rs
    for j in range(ROWS_PT // CH):
        pltpu.sync_copy(acc_sh.at[pl.ds(rs + j * CH, CH)], rows_v)
        pltpu.sync_copy(rows_v, out.at[pl.ds(obase + j * CH, CH)])


_RB = 1024


def _invcnt_body(cntp_ref, out_ref):
    c = jnp.maximum(cntp_ref[0][:, 0:1] + cntp_ref[1][:, 0:1], 1.0)
    out_ref[...] = 1.0 / c


def _invcnt(cntp):
    return pl.pallas_call(
        _invcnt_body,
        grid=(N_PAD // _RB,),
        in_specs=[pl.BlockSpec((NC, _RB, D), lambda i: (0, i, 0))],
        out_specs=pl.BlockSpec((_RB, 1), lambda i: (i, 0)),
        out_shape=jax.ShapeDtypeStruct((N_PAD, 1), jnp.float32),
    )(cntp)


def _linear_body(parts_ref, inv_ref, x_ref, wl_ref, wr_ref, b_ref, out_ref,
                 *, relu):
    mean = (parts_ref[0] + parts_ref[1]) * inv_ref[...]
    h = (jnp.dot(mean, wl_ref[...], preferred_element_type=jnp.float32)
         + jnp.dot(x_ref[...], wr_ref[...], preferred_element_type=jnp.float32)
         + b_ref[...])
    out_ref[...] = jnp.maximum(h, 0.0) if relu else h


def _linear(parts, inv, x, wlT, wrT, b, relu):
    return pl.pallas_call(
        functools.partial(_linear_body, relu=relu),
        grid=(N_PAD // _RB,),
        in_specs=[
            pl.BlockSpec((NC, _RB, D), lambda i: (0, i, 0)),
            pl.BlockSpec((_RB, 1), lambda i: (i, 0)),
            pl.BlockSpec((_RB, D), lambda i: (i, 0)),
            pl.BlockSpec((D, D), lambda i: (0, 0)),
            pl.BlockSpec((D, D), lambda i: (0, 0)),
            pl.BlockSpec((1, D), lambda i: (0, 0)),
        ],
        out_specs=pl.BlockSpec((_RB, D), lambda i: (i, 0)),
        out_shape=jax.ShapeDtypeStruct((N_PAD, D), jnp.float32),
    )(parts, inv, x, wlT, wrT, b)


def kernel(x, edge_index, W1l, W1r, b1, W2l, W2r, b2, W3l, W3r, b3):
    src = edge_index[0].astype(jnp.int32)
    dst = edge_index[1].astype(jnp.int32)
    xp = jnp.pad(x, ((0, N_PAD - N_NODES), (0, 0)))
    z128 = jnp.zeros((N_PAD, D), jnp.float32)
    ones = jnp.ones((CH, D), jnp.float32)

    w3l = jnp.pad(W3l.T, ((0, 0), (0, D - W3l.shape[0])))
    w3r = jnp.pad(W3r.T, ((0, 0), (0, D - W3r.shape[0])))
    b3p = jnp.pad(b3, (0, D - b3.shape[0]))

    cntp = _cnt128(dst, z128, ones)
    inv = _invcnt(cntp.reshape(NC, N_PAD, D))

    parts1 = _segsum(xp, src, dst, z128).reshape(NC, N_PAD, D)
    h1 = _linear(parts1, inv, xp, W1l.T, W1r.T, b1.reshape(1, D), relu=True)
    parts2 = _segsum(h1, src, dst, z128).reshape(NC, N_PAD, D)
    h2 = _linear(parts2, inv, h1, W2l.T, W2r.T, b2.reshape(1, D), relu=True)
    parts3 = _segsum(h2, src, dst, z128).reshape(NC, N_PAD, D)
    out = _linear(parts3, inv, h2, w3l, w3r, b3p.reshape(1, D), relu=False)
    return out[:N_NODES, :W3l.shape[0]]
